# baseline (speedup 1.0000x reference)
import functools

import jax
import jax.numpy as jnp
from jax import lax
from jax.experimental import pallas as pl
from jax.experimental.pallas import tpu as pltpu
from jax.experimental.pallas import tpu_sc as plsc

B = 16384
D = 64
HALF = 32
NC, NS = 2, 16
NW = NC * NS
B_PER_W = B // NW
CHUNK = 128
NCHUNK = B_PER_W // CHUNK

_mesh = plsc.VectorSubcoreMesh(core_axis_name="c", subcore_axis_name="s")


@functools.partial(
    pl.kernel,
    mesh=_mesh,
    out_type=jax.ShapeDtypeStruct((B, D), jnp.float32),
    scratch_types=[
        pltpu.VMEM((NCHUNK, CHUNK), jnp.int32),
        pltpu.VMEM((B_PER_W, D), jnp.float32),
        pltpu.SemaphoreType.DMA,
    ],
    compiler_params=pltpu.CompilerParams(use_tc_tiling_on_sc=False),
)
def _sc_gather(idx_hbm, table_hbm, out_hbm, idx_v, rows_v, sem):
    wid = lax.axis_index("s") * NC + lax.axis_index("c")
    pltpu.sync_copy(idx_hbm.at[wid], idx_v)
    copies = [
        pltpu.async_copy(
            table_hbm.at[idx_v.at[j]],
            rows_v.at[pl.ds(j * CHUNK, CHUNK)],
            sem,
        )
        for j in range(NCHUNK)
    ]
    for c in copies:
        c.wait()
    pltpu.sync_copy(rows_v, out_hbm.at[pl.ds(wid * B_PER_W, B_PER_W)])


BLK = 2048
GRID = B // BLK


def _tc_body(g_ref, bkt_ref, wbT_ref, bb_ref, weT_ref, wb2T_ref, bc_ref, out_ref):
    f = jnp.dot(bkt_ref[...], wbT_ref[...], preferred_element_type=jnp.float32)
    f = f + bb_ref[...]
    y = jnp.dot(g_ref[...], weT_ref[...], preferred_element_type=jnp.float32)
    y = y + jnp.dot(f, wb2T_ref[...], preferred_element_type=jnp.float32)
    out_ref[...] = jnp.tanh(y + bc_ref[...])


_tc_dense = pl.pallas_call(
    _tc_body,
    grid=(GRID,),
    in_specs=[
        pl.BlockSpec((BLK, D), lambda i: (i, 0)),
        pl.BlockSpec((BLK, 4), lambda i: (i, 0)),
        pl.BlockSpec((4, HALF), lambda i: (0, 0)),
        pl.BlockSpec((1, HALF), lambda i: (0, 0)),
        pl.BlockSpec((D, D), lambda i: (0, 0)),
        pl.BlockSpec((HALF, D), lambda i: (0, 0)),
        pl.BlockSpec((1, D), lambda i: (0, 0)),
    ],
    out_specs=pl.BlockSpec((BLK, D), lambda i: (i, 0)),
    out_shape=jax.ShapeDtypeStruct((B, D), jnp.float32),
)


def kernel(skill_ids, bkt_params, table, W_bkt, b_bkt, W_comb, b_comb):
    idx = skill_ids.astype(jnp.int32).reshape(NW, NCHUNK, CHUNK)
    gathered = _sc_gather(idx, table)
    return _tc_dense(
        gathered,
        bkt_params,
        W_bkt.T,
        b_bkt.reshape(1, HALF),
        W_comb[:, :D].T,
        W_comb[:, D:].T,
        b_comb.reshape(1, D),
    )
